# Initial kernel scaffold; baseline (speedup 1.0000x reference)
import functools

import jax
import jax.numpy as jnp
from jax import lax
from jax.experimental import pallas as pl
from jax.experimental.pallas import tpu as pltpu
from jax.experimental.pallas import tpu_sc as plsc

N_NODES = 10000
N_EDGES = 320000
NC = 2
NS = 16
NW = NC * NS
LANES = 128
EPT = N_EDGES // NW
CH = -(-EPT // LANES)
EPT_PAD = CH * LANES
NPAD = 10240
RPT = NPAD // NS

_mesh = plsc.VectorSubcoreMesh(core_axis_name="c", subcore_axis_name="s")


@functools.partial(
    pl.kernel,
    out_type=jax.ShapeDtypeStruct((NC, NPAD, 16), jnp.float32),
    mesh=_mesh,
    scratch_types=[
        pltpu.VMEM((CH, LANES), jnp.int32),
        pltpu.VMEM((LANES, 16), jnp.float32),
        pltpu.VMEM_SHARED((NPAD, 16), jnp.float32),
    ],
)
def _sc_degree(dst_hbm, ones_hbm, zeros_hbm, degp_hbm, dst_v, ones_v, acc_sh):
    c = lax.axis_index("c")
    s = lax.axis_index("s")
    pltpu.sync_copy(dst_hbm.at[c].at[s], dst_v)
    pltpu.sync_copy(ones_hbm, ones_v)
    row0 = s * RPT
    pltpu.sync_copy(zeros_hbm.at[pl.ds(row0, RPT), 0:16],
                    acc_sh.at[pl.ds(row0, RPT)])
    plsc.subcore_barrier()

    def body(j, carry):
        pltpu.sync_copy(ones_v, acc_sh.at[dst_v.at[j]], add=True)
        return carry

    lax.fori_loop(0, CH, body, 0, unroll=False)
    plsc.subcore_barrier()
    pltpu.sync_copy(acc_sh.at[pl.ds(row0, RPT)],
                    degp_hbm.at[c].at[pl.ds(row0, RPT)])


@functools.partial(
    pl.kernel,
    out_type=jax.ShapeDtypeStruct((NC, NPAD, 128), jnp.float32),
    mesh=_mesh,
    scratch_types=[
        pltpu.VMEM((CH, LANES), jnp.int32),
        pltpu.VMEM((CH, LANES), jnp.int32),
        pltpu.VMEM((LANES, 128), jnp.float32),
        pltpu.VMEM_SHARED((NPAD, 128), jnp.float32),
        pltpu.SemaphoreType.DMA,
    ],
)
def _sc_aggregate(src_hbm, dst_hbm, y_hbm, zeros_hbm, zp_hbm,
                  src_v, dst_v, rows_v, acc_sh, sem):
    c = lax.axis_index("c")
    s = lax.axis_index("s")
    pltpu.sync_copy(src_hbm.at[c].at[s], src_v)
    pltpu.sync_copy(dst_hbm.at[c].at[s], dst_v)
    row0 = s * RPT
    pltpu.sync_copy(zeros_hbm.at[pl.ds(row0, RPT)], acc_sh.at[pl.ds(row0, RPT)])
    plsc.subcore_barrier()

    def body(j, carry):
        pltpu.async_copy(y_hbm.at[src_v.at[j]], rows_v, sem).wait()
        pltpu.sync_copy(rows_v, acc_sh.at[dst_v.at[j]], add=True)
        return carry

    lax.fori_loop(0, CH, body, 0, unroll=False)
    plsc.subcore_barrier()
    pltpu.sync_copy(acc_sh.at[pl.ds(row0, RPT)],
                    zp_hbm.at[c].at[pl.ds(row0, RPT)])


def _tc_prep_body(degp_ref, x_ref, dinv_ref, y1_ref):
    deg = degp_ref[0] + degp_ref[1] + 1.0
    dinv = lax.rsqrt(deg)
    dinv_ref[...] = dinv
    d = jnp.broadcast_to(dinv[:N_NODES, 0:1], (N_NODES, 128))
    y1_ref[...] = x_ref[...] * d


def _tc_mid_body(zp_ref, y1_ref, dinv_ref, w1_ref, b1_ref, y2_ref):
    d = jnp.broadcast_to(dinv_ref[:N_NODES, 0:1], (N_NODES, 128))
    ax = d * (zp_ref[0, :N_NODES, :] + zp_ref[1, :N_NODES, :] + y1_ref[...])
    h = jnp.maximum(
        jnp.dot(ax, w1_ref[...], preferred_element_type=jnp.float32)
        + b1_ref[...], 0.0)
    y2_ref[...] = h * d


def _tc_head_body(zp_ref, y2_ref, dinv_ref, wmu_ref, bmu_ref, wls_ref,
                  bls_ref, mu_ref, ls_ref):
    d = jnp.broadcast_to(dinv_ref[:N_NODES, 0:1], (N_NODES, 128))
    ah = d * (zp_ref[0, :N_NODES, :] + zp_ref[1, :N_NODES, :] + y2_ref[...])
    mu_ref[...] = (
        jnp.dot(ah, wmu_ref[...], preferred_element_type=jnp.float32)
        + bmu_ref[...])
    ls_ref[...] = (
        jnp.dot(ah, wls_ref[...], preferred_element_type=jnp.float32)
        + bls_ref[...])


def kernel(x, edge_index, W1, b1, Wmu, bmu, Wls, bls):
    src = edge_index[0].astype(jnp.int32)
    dst = edge_index[1].astype(jnp.int32)
    pad = NW * EPT_PAD - N_EDGES
    srcp = jnp.concatenate([src, jnp.zeros((pad,), jnp.int32)])
    dstp = jnp.concatenate([dst, jnp.full((pad,), N_NODES, jnp.int32)])
    srcp = srcp.reshape(NC, NS, CH, LANES)
    dstp = dstp.reshape(NC, NS, CH, LANES)

    ones16 = jnp.ones((LANES, 16), jnp.float32)
    zeros128 = jnp.zeros((NPAD, 128), jnp.float32)

    degp = _sc_degree(dstp, ones16, zeros128)

    dinv, y1 = pl.pallas_call(
        _tc_prep_body,
        out_shape=[
            jax.ShapeDtypeStruct((NPAD, 16), jnp.float32),
            jax.ShapeDtypeStruct((N_NODES, 128), jnp.float32),
        ],
    )(degp, x)

    zp1 = _sc_aggregate(srcp, dstp, y1, zeros128)

    y2 = pl.pallas_call(
        _tc_mid_body,
        out_shape=jax.ShapeDtypeStruct((N_NODES, 128), jnp.float32),
    )(zp1, y1, dinv, W1, b1.reshape(1, 128))

    zp2 = _sc_aggregate(srcp, dstp, y2, zeros128)

    mu, logstd = pl.pallas_call(
        _tc_head_body,
        out_shape=[
            jax.ShapeDtypeStruct((N_NODES, 64), jnp.float32),
            jax.ShapeDtypeStruct((N_NODES, 64), jnp.float32),
        ],
    )(zp2, y2, dinv, Wmu, bmu.reshape(1, 64), Wls, bls.reshape(1, 64))

    return (mu, logstd)


# speedup vs baseline: 16.1141x; 16.1141x over previous
import functools

import jax
import jax.numpy as jnp
from jax import lax
from jax.experimental import pallas as pl
from jax.experimental.pallas import tpu as pltpu
from jax.experimental.pallas import tpu_sc as plsc

N_NODES = 10000
N_EDGES = 320000
NC = 2
NS = 16
NW = NC * NS
LANES = 128
EPT = N_EDGES // NW
CH = -(-EPT // LANES)
EPT_PAD = CH * LANES
NPAD = 10240
RPT = NPAD // NS

_mesh = plsc.VectorSubcoreMesh(core_axis_name="c", subcore_axis_name="s")


@functools.partial(
    pl.kernel,
    out_type=jax.ShapeDtypeStruct((NC, NPAD, 16), jnp.float32),
    mesh=_mesh,
    scratch_types=[
        pltpu.VMEM((CH, LANES), jnp.int32),
        pltpu.VMEM((LANES, 16), jnp.float32),
        pltpu.VMEM_SHARED((NPAD, 16), jnp.float32),
    ],
)
def _sc_degree(dst_hbm, ones_hbm, zeros_hbm, degp_hbm, dst_v, ones_v, acc_sh):
    c = lax.axis_index("c")
    s = lax.axis_index("s")
    pltpu.sync_copy(dst_hbm.at[c].at[s], dst_v)
    pltpu.sync_copy(ones_hbm, ones_v)
    row0 = s * RPT
    pltpu.sync_copy(zeros_hbm.at[pl.ds(row0, RPT)],
                    acc_sh.at[pl.ds(row0, RPT)])
    plsc.subcore_barrier()

    def body(j, carry):
        pltpu.sync_copy(ones_v, acc_sh.at[dst_v.at[j]], add=True)
        return carry

    lax.fori_loop(0, CH, body, 0, unroll=False)
    plsc.subcore_barrier()
    pltpu.sync_copy(acc_sh.at[pl.ds(row0, RPT)],
                    degp_hbm.at[c].at[pl.ds(row0, RPT)])


@functools.partial(
    pl.kernel,
    out_type=jax.ShapeDtypeStruct((NC, NPAD, 128), jnp.float32),
    mesh=_mesh,
    scratch_types=[
        pltpu.VMEM((CH, LANES), jnp.int32),
        pltpu.VMEM((CH, LANES), jnp.int32),
        pltpu.VMEM((LANES, 128), jnp.float32),
        pltpu.VMEM_SHARED((NPAD, 128), jnp.float32),
        pltpu.SemaphoreType.DMA,
    ],
)
def _sc_aggregate(src_hbm, dst_hbm, y_hbm, zeros_hbm, zp_hbm,
                  src_v, dst_v, rows_v, acc_sh, sem):
    c = lax.axis_index("c")
    s = lax.axis_index("s")
    pltpu.sync_copy(src_hbm.at[c].at[s], src_v)
    pltpu.sync_copy(dst_hbm.at[c].at[s], dst_v)
    row0 = s * RPT
    pltpu.sync_copy(zeros_hbm.at[pl.ds(row0, RPT)], acc_sh.at[pl.ds(row0, RPT)])
    plsc.subcore_barrier()

    def body(j, carry):
        pltpu.async_copy(y_hbm.at[src_v.at[j]], rows_v, sem).wait()
        pltpu.sync_copy(rows_v, acc_sh.at[dst_v.at[j]], add=True)
        return carry

    lax.fori_loop(0, CH, body, 0, unroll=False)
    plsc.subcore_barrier()
    pltpu.sync_copy(acc_sh.at[pl.ds(row0, RPT)],
                    zp_hbm.at[c].at[pl.ds(row0, RPT)])


def _tc_prep_body(degp_ref, x_ref, dinv_ref, y1_ref):
    deg = degp_ref[0] + degp_ref[1] + 1.0
    dinv = lax.rsqrt(deg)
    dinv_ref[...] = dinv
    d = jnp.broadcast_to(dinv[:N_NODES, 0:1], (N_NODES, 128))
    y1_ref[...] = x_ref[...] * d


def _tc_mid_body(zp_ref, y1_ref, dinv_ref, w1_ref, b1_ref, y2_ref):
    d = jnp.broadcast_to(dinv_ref[:N_NODES, 0:1], (N_NODES, 128))
    ax = d * (zp_ref[0, :N_NODES, :] + zp_ref[1, :N_NODES, :] + y1_ref[...])
    h = jnp.maximum(
        jnp.dot(ax, w1_ref[...], preferred_element_type=jnp.float32)
        + b1_ref[...], 0.0)
    y2_ref[...] = h * d


def _tc_head_body(zp_ref, y2_ref, dinv_ref, wmu_ref, bmu_ref, wls_ref,
                  bls_ref, mu_ref, ls_ref):
    d = jnp.broadcast_to(dinv_ref[:N_NODES, 0:1], (N_NODES, 128))
    ah = d * (zp_ref[0, :N_NODES, :] + zp_ref[1, :N_NODES, :] + y2_ref[...])
    mu_ref[...] = (
        jnp.dot(ah, wmu_ref[...], preferred_element_type=jnp.float32)
        + bmu_ref[...])
    ls_ref[...] = (
        jnp.dot(ah, wls_ref[...], preferred_element_type=jnp.float32)
        + bls_ref[...])


def kernel(x, edge_index, W1, b1, Wmu, bmu, Wls, bls):
    src = edge_index[0].astype(jnp.int32)
    dst = edge_index[1].astype(jnp.int32)
    pad = NW * EPT_PAD - N_EDGES
    srcp = jnp.concatenate([src, jnp.zeros((pad,), jnp.int32)])
    dstp = jnp.concatenate([dst, jnp.full((pad,), N_NODES, jnp.int32)])
    srcp = srcp.reshape(NC, NS, CH, LANES)
    dstp = dstp.reshape(NC, NS, CH, LANES)

    ones16 = jnp.ones((LANES, 16), jnp.float32)
    zeros16 = jnp.zeros((NPAD, 16), jnp.float32)
    zeros128 = jnp.zeros((NPAD, 128), jnp.float32)

    degp = _sc_degree(dstp, ones16, zeros16)

    dinv, y1 = pl.pallas_call(
        _tc_prep_body,
        out_shape=[
            jax.ShapeDtypeStruct((NPAD, 16), jnp.float32),
            jax.ShapeDtypeStruct((N_NODES, 128), jnp.float32),
        ],
    )(degp, x)

    zp1 = _sc_aggregate(srcp, dstp, y1, zeros128)

    y2 = pl.pallas_call(
        _tc_mid_body,
        out_shape=jax.ShapeDtypeStruct((N_NODES, 128), jnp.float32),
    )(zp1, y1, dinv, W1, b1.reshape(1, 128))

    zp2 = _sc_aggregate(srcp, dstp, y2, zeros128)

    mu, logstd = pl.pallas_call(
        _tc_head_body,
        out_shape=[
            jax.ShapeDtypeStruct((N_NODES, 64), jnp.float32),
            jax.ShapeDtypeStruct((N_NODES, 64), jnp.float32),
        ],
    )(zp2, y2, dinv, Wmu, bmu.reshape(1, 64), Wls, bls.reshape(1, 64))

    return (mu, logstd)
